# baseline (speedup 1.0000x reference)
import functools

import jax
import jax.numpy as jnp
from jax import lax
from jax.experimental import pallas as pl
from jax.experimental.pallas import tpu as pltpu
from jax.experimental.pallas import tpu_sc as plsc

N_NODES = 10000
N_EDGES = 320000
D_IN = 128
HID = 128
H3 = 384
OUT = 64

NC = 2
NS = 16
NW = NC * NS
CHUNK = 64
NBUF = 4
K_STAGE = 40
K_CHUNKS = 160
E_PAD = NW * K_CHUNKS * CHUNK
ACC_ROWS = 10112
TRASH = N_NODES
ZROWS = ACC_ROWS // NS
DEG_W = 128

BLK = 2000
EPS = 1e-5


def _sc_mesh():
    return plsc.VectorSubcoreMesh(
        core_axis_name="c", subcore_axis_name="s",
        num_cores=NC, num_subcores=NS)


def _make_sc_spmm():
    @functools.partial(
        pl.kernel,
        out_type=jax.ShapeDtypeStruct((NC * ACC_ROWS, HID), jnp.float32),
        mesh=_sc_mesh(),
        scratch_types=[
            pltpu.VMEM((K_STAGE, CHUNK), jnp.int32),
            pltpu.VMEM((K_STAGE, CHUNK), jnp.int32),
            [pltpu.VMEM((CHUNK, HID), jnp.float32)] * NBUF,
            [pltpu.SemaphoreType.DMA] * NBUF,
            [pltpu.SemaphoreType.DMA] * NBUF,
            pltpu.VMEM_SHARED((ACC_ROWS, HID), jnp.float32),
        ],
    )
    def spmm(xt, row3, col3, zeros, out, row_v, col_v, bufs, gsems, ssems,
             acc):
        c = lax.axis_index("c")
        s = lax.axis_index("s")
        wid = s * NC + c
        pltpu.sync_copy(zeros.at[pl.ds(s * ZROWS, ZROWS)],
                        acc.at[pl.ds(s * ZROWS, ZROWS)])
        plsc.subcore_barrier()

        def g_issue(b, j):
            pltpu.async_copy(xt.at[row_v.at[j]], bufs[b], gsems[b])

        def g_wait(b, j):
            pltpu.make_async_copy(xt.at[row_v.at[j]], bufs[b],
                                  gsems[b]).wait()

        def s_issue(b, j):
            pltpu.async_copy(bufs[b], acc.at[col_v.at[j]], ssems[b],
                             add=True)

        def s_wait(b, j):
            pltpu.make_async_copy(bufs[b], acc.at[col_v.at[j]],
                                  ssems[b]).wait()

        K = K_STAGE
        for stage in range(K_CHUNKS // K_STAGE):
            pltpu.sync_copy(row3.at[wid, pl.ds(stage * K, K)], row_v)
            pltpu.sync_copy(col3.at[wid, pl.ds(stage * K, K)], col_v)
            g_issue(0, 0)
            for b in range(NBUF):
                if b == NBUF - 1:
                    s_wait(0, 0)
                    g_issue(0, NBUF)
                else:
                    g_issue(b + 1, b + 1)
                g_wait(b, b)
                s_issue(b, b)

            def ring(jj, carry):
                for b in range(NBUF):
                    j = jj * NBUF + b
                    bn = (b + 1) % NBUF
                    s_wait(bn, j + 1 - NBUF)
                    g_issue(bn, (j + 1) % K)
                    g_wait(b, j)
                    s_issue(b, j)
                return carry

            lax.fori_loop(1, K // NBUF, ring, 0)
            for b in range(1, NBUF):
                s_wait(b, K - NBUF + b)
            g_wait(0, 0)
        plsc.subcore_barrier()
        pltpu.sync_copy(
            acc.at[pl.ds(s * ZROWS, ZROWS)],
            out.at[pl.ds(c * ACC_ROWS + s * ZROWS, ZROWS)])

    return spmm


def _make_sc_deg():
    @functools.partial(
        pl.kernel,
        out_type=jax.ShapeDtypeStruct((NC * ACC_ROWS, DEG_W), jnp.float32),
        mesh=_sc_mesh(),
        scratch_types=[
            pltpu.VMEM((K_CHUNKS, CHUNK), jnp.int32),
            pltpu.VMEM((CHUNK, DEG_W), jnp.float32),
            pltpu.VMEM_SHARED((ACC_ROWS, DEG_W), jnp.float32),
            pltpu.SemaphoreType.DMA,
        ],
    )
    def degk(col3, zeros, ones, out, col_v, ones_v, acc, sem):
        c = lax.axis_index("c")
        s = lax.axis_index("s")
        wid = s * NC + c
        pltpu.sync_copy(zeros.at[pl.ds(s * ZROWS, ZROWS)],
                        acc.at[pl.ds(s * ZROWS, ZROWS)])
        pltpu.sync_copy(col3.at[wid], col_v)
        pltpu.sync_copy(ones, ones_v)
        plsc.subcore_barrier()

        def group(jj, carry):
            base = jj * 8
            sds = [pltpu.async_copy(ones_v, acc.at[col_v.at[base + b]],
                                    sem, add=True) for b in range(8)]
            for sd in sds:
                sd.wait()
            return carry

        lax.fori_loop(0, K_CHUNKS // 8, group, 0)
        plsc.subcore_barrier()
        pltpu.sync_copy(
            acc.at[pl.ds(s * ZROWS, ZROWS)],
            out.at[pl.ds(c * ACC_ROWS + s * ZROWS, ZROWS)])

    return degk


_sc_spmm = _make_sc_spmm()
_sc_deg = _make_sc_deg()


def _tc_a_first(h, W, deg):
    def body(h_ref, w_ref, deg_ref, o0, o1, o2):
        t = jnp.dot(h_ref[...], w_ref[...], preferred_element_type=jnp.float32)
        dis = lax.rsqrt(deg_ref[...])
        o0[...] = t[:, :HID]
        o1[...] = dis * t[:, HID:2 * HID]
        o2[...] = dis * t[:, 2 * HID:]

    grid = (N_NODES // BLK,)
    return pl.pallas_call(
        body,
        grid=grid,
        in_specs=[
            pl.BlockSpec((BLK, D_IN), lambda i: (i, 0)),
            pl.BlockSpec((D_IN, H3), lambda i: (0, 0)),
            pl.BlockSpec((BLK, 1), lambda i: (i, 0)),
        ],
        out_specs=[
            pl.BlockSpec((BLK, HID), lambda i: (i, 0)),
            pl.BlockSpec((BLK, HID), lambda i: (i, 0)),
            pl.BlockSpec((BLK, HID), lambda i: (i, 0)),
        ],
        out_shape=[jax.ShapeDtypeStruct((N_NODES, HID), jnp.float32)] * 3,
    )(h, W, deg)


def _tc_a_bn(pre, sums, g, b, W, deg):
    def body(p_ref, s_ref, g_ref, b_ref, w_ref, deg_ref, o0, o1, o2):
        mu = s_ref[0:1, :] * (1.0 / N_NODES)
        var = s_ref[1:2, :] * (1.0 / N_NODES) - mu * mu
        inv = lax.rsqrt(var + EPS)
        h = (p_ref[...] - mu) * inv * g_ref[...] + b_ref[...]
        t = jnp.dot(h, w_ref[...], preferred_element_type=jnp.float32)
        dis = lax.rsqrt(deg_ref[...])
        o0[...] = t[:, :HID]
        o1[...] = dis * t[:, HID:2 * HID]
        o2[...] = dis * t[:, 2 * HID:]

    grid = (N_NODES // BLK,)
    return pl.pallas_call(
        body,
        grid=grid,
        in_specs=[
            pl.BlockSpec((BLK, H3), lambda i: (i, 0)),
            pl.BlockSpec((2, H3), lambda i: (0, 0)),
            pl.BlockSpec((1, H3), lambda i: (0, 0)),
            pl.BlockSpec((1, H3), lambda i: (0, 0)),
            pl.BlockSpec((H3, H3), lambda i: (0, 0)),
            pl.BlockSpec((BLK, 1), lambda i: (i, 0)),
        ],
        out_specs=[
            pl.BlockSpec((BLK, HID), lambda i: (i, 0)),
            pl.BlockSpec((BLK, HID), lambda i: (i, 0)),
            pl.BlockSpec((BLK, HID), lambda i: (i, 0)),
        ],
        out_shape=[jax.ShapeDtypeStruct((N_NODES, HID), jnp.float32)] * 3,
    )(pre, sums, g, b, W, deg)


def _tc_comb(sa, sb, u, deg):
    def body(a_ref, b_ref, u_ref, deg_ref, o_ref):
        o_ref[...] = (a_ref[...] + b_ref[...] + u_ref[...]) / deg_ref[...]

    grid = (N_NODES // BLK,)
    bs = pl.BlockSpec((BLK, HID), lambda i: (i, 0))
    return pl.pallas_call(
        body,
        grid=grid,
        in_specs=[bs, bs, bs, pl.BlockSpec((BLK, 1), lambda i: (i, 0))],
        out_specs=bs,
        out_shape=jax.ShapeDtypeStruct((N_NODES, HID), jnp.float32),
    )(sa, sb, u, deg)


def _tc_c(o0, s1a, s1b, u1, s2a, s2b, u2b, bias, deg):
    def body(o0_ref, a1, b1, u1_ref, a2, b2, u2_ref, bias_ref, deg_ref,
             pre_ref, sums_ref):
        i = pl.program_id(0)
        dis = lax.rsqrt(deg_ref[...])
        out1 = dis * (a1[...] + b1[...] + u1_ref[...])
        out2 = dis * (a2[...] + b2[...] + u2_ref[...])
        pre = jnp.concatenate([o0_ref[...], out1, out2], axis=1) + bias_ref[...]
        pre_ref[...] = pre
        cs = jnp.concatenate(
            [jnp.sum(pre, axis=0, keepdims=True),
             jnp.sum(pre * pre, axis=0, keepdims=True)], axis=0)
        sums_ref[...] = jnp.where(i == 0, cs, sums_ref[...] + cs)

    grid = (N_NODES // BLK,)
    bs = pl.BlockSpec((BLK, HID), lambda i: (i, 0))
    return pl.pallas_call(
        body,
        grid=grid,
        in_specs=[bs, bs, bs, bs, bs, bs, bs,
                  pl.BlockSpec((1, H3), lambda i: (0, 0)),
                  pl.BlockSpec((BLK, 1), lambda i: (i, 0))],
        out_specs=[pl.BlockSpec((BLK, H3), lambda i: (i, 0)),
                   pl.BlockSpec((2, H3), lambda i: (0, 0))],
        out_shape=[jax.ShapeDtypeStruct((N_NODES, H3), jnp.float32),
                   jax.ShapeDtypeStruct((2, H3), jnp.float32)],
    )(o0, s1a, s1b, u1, s2a, s2b, u2b, bias, deg)


def _tc_head(pre, sums, g, b, lw, lb):
    def body(p_ref, s_ref, g_ref, b_ref, w_ref, lb_ref, o_ref):
        mu = s_ref[0:1, :] * (1.0 / N_NODES)
        var = s_ref[1:2, :] * (1.0 / N_NODES) - mu * mu
        inv = lax.rsqrt(var + EPS)
        h = (p_ref[...] - mu) * inv * g_ref[...] + b_ref[...]
        logits = jnp.dot(h, w_ref[...], preferred_element_type=jnp.float32)
        logits = logits + lb_ref[...]
        m = jnp.max(logits, axis=1, keepdims=True)
        z = logits - m
        lse = jnp.log(jnp.sum(jnp.exp(z), axis=1, keepdims=True))
        o_ref[...] = z - lse

    grid = (N_NODES // BLK,)
    return pl.pallas_call(
        body,
        grid=grid,
        in_specs=[
            pl.BlockSpec((BLK, H3), lambda i: (i, 0)),
            pl.BlockSpec((2, H3), lambda i: (0, 0)),
            pl.BlockSpec((1, H3), lambda i: (0, 0)),
            pl.BlockSpec((1, H3), lambda i: (0, 0)),
            pl.BlockSpec((H3, OUT), lambda i: (0, 0)),
            pl.BlockSpec((1, OUT), lambda i: (0, 0)),
        ],
        out_specs=pl.BlockSpec((BLK, OUT), lambda i: (i, 0)),
        out_shape=jax.ShapeDtypeStruct((N_NODES, OUT), jnp.float32),
    )(pre, sums, g, b, lw, lb)


def kernel(x, edge_index, params):
    p = params
    row = edge_index[0]
    col = edge_index[1]
    npad = E_PAD - N_EDGES
    pad_r = (jnp.arange(npad, dtype=jnp.int32) % 128)
    pad_c = TRASH + (jnp.arange(npad, dtype=jnp.int32) % (ACC_ROWS - TRASH))
    row3 = (jnp.concatenate([row, pad_r])
            .reshape(K_CHUNKS, NW, CHUNK).transpose(1, 0, 2))
    col3 = (jnp.concatenate([col, pad_c])
            .reshape(K_CHUNKS, NW, CHUNK).transpose(1, 0, 2))
    zeros128 = jnp.zeros((ACC_ROWS, HID), jnp.float32)
    zeros16 = jnp.zeros((ACC_ROWS, DEG_W), jnp.float32)
    ones16 = jnp.ones((CHUNK, DEG_W), jnp.float32)

    degp = _sc_deg(col3, zeros16, ones16)
    deg = (degp[:N_NODES, 0]
           + degp[ACC_ROWS:ACC_ROWS + N_NODES, 0] + 1.0).reshape(N_NODES, 1)

    def S(u):
        sp = _sc_spmm(u, row3, col3, zeros128)
        return sp[:N_NODES], sp[ACC_ROWS:ACC_ROWS + N_NODES]

    W1 = jnp.concatenate([p["c1_w0"], p["c1_w1"], p["c1_w2"]], axis=1)
    out0, u1, u2 = _tc_a_first(x, W1, deg)
    s1a, s1b = S(u1)
    s2a, s2b = S(u2)
    u2b = _tc_comb(s2a, s2b, u2, deg)
    s2ba, s2bb = S(u2b)
    pre1, sums1 = _tc_c(out0, s1a, s1b, u1, s2ba, s2bb, u2b,
                        p["c1_b"].reshape(1, H3), deg)

    pre, sums = pre1, sums1
    for l, nk in (("c2", "n1"), ("c3", "n2")):
        W = jnp.concatenate([p[l + "_w0"], p[l + "_w1"], p[l + "_w2"]], axis=1)
        out0, u1, u2 = _tc_a_bn(pre, sums, p[nk + "_g"].reshape(1, H3),
                                p[nk + "_b"].reshape(1, H3), W, deg)
        s1a, s1b = S(u1)
        s2a, s2b = S(u2)
        u2b = _tc_comb(s2a, s2b, u2, deg)
        s2ba, s2bb = S(u2b)
        pre, sums = _tc_c(out0, s1a, s1b, u1, s2ba, s2bb, u2b,
                          p[l + "_b"].reshape(1, H3), deg)

    return _tc_head(pre, sums, p["n3_g"].reshape(1, H3),
                    p["n3_b"].reshape(1, H3), p["lin_w"],
                    p["lin_b"].reshape(1, OUT))


# speedup vs baseline: 1.2386x; 1.2386x over previous
import functools

import jax
import jax.numpy as jnp
from jax import lax
from jax.experimental import pallas as pl
from jax.experimental.pallas import tpu as pltpu
from jax.experimental.pallas import tpu_sc as plsc

N_NODES = 10000
N_EDGES = 320000
D_IN = 128
HID = 128
H3 = 384
OUT = 64

NC = 2
NS = 16
NW = NC * NS
CHUNK = 128
NBUF = 2
K_STAGE = 40
K_CHUNKS = 80
E_PAD = NW * K_CHUNKS * CHUNK
ACC_ROWS = 10112
TRASH = N_NODES
ZROWS = ACC_ROWS // NS
DEG_W = 128

BLK = 2000
EPS = 1e-5


def _sc_mesh():
    return plsc.VectorSubcoreMesh(
        core_axis_name="c", subcore_axis_name="s",
        num_cores=NC, num_subcores=NS)


def _make_sc_spmm():
    @functools.partial(
        pl.kernel,
        out_type=jax.ShapeDtypeStruct((NC * ACC_ROWS, HID), jnp.float32),
        mesh=_sc_mesh(),
        scratch_types=[
            pltpu.VMEM((K_STAGE, CHUNK), jnp.int32),
            pltpu.VMEM((K_STAGE, CHUNK), jnp.int32),
            [pltpu.VMEM((CHUNK, HID), jnp.float32)] * NBUF,
            [pltpu.SemaphoreType.DMA] * NBUF,
            [pltpu.SemaphoreType.DMA] * NBUF,
            pltpu.VMEM_SHARED((ACC_ROWS, HID), jnp.float32),
        ],
    )
    def spmm(xt, row3, col3, zeros, out, row_v, col_v, bufs, gsems, ssems,
             acc):
        c = lax.axis_index("c")
        s = lax.axis_index("s")
        wid = s * NC + c
        pltpu.sync_copy(zeros.at[pl.ds(s * ZROWS, ZROWS)],
                        acc.at[pl.ds(s * ZROWS, ZROWS)])
        plsc.subcore_barrier()

        def g_issue(b, j):
            pltpu.async_copy(xt.at[row_v.at[j]], bufs[b], gsems[b])

        def g_wait(b, j):
            pltpu.make_async_copy(xt.at[row_v.at[j]], bufs[b],
                                  gsems[b]).wait()

        def s_issue(b, j):
            pltpu.async_copy(bufs[b], acc.at[col_v.at[j]], ssems[b],
                             add=True)

        def s_wait(b, j):
            pltpu.make_async_copy(bufs[b], acc.at[col_v.at[j]],
                                  ssems[b]).wait()

        K = K_STAGE
        for stage in range(K_CHUNKS // K_STAGE):
            pltpu.sync_copy(row3.at[wid, pl.ds(stage * K, K)], row_v)
            pltpu.sync_copy(col3.at[wid, pl.ds(stage * K, K)], col_v)
            g_issue(0, 0)
            for b in range(NBUF):
                if b == NBUF - 1:
                    s_wait(0, 0)
                    g_issue(0, NBUF)
                else:
                    g_issue(b + 1, b + 1)
                g_wait(b, b)
                s_issue(b, b)

            def ring(jj, carry):
                for b in range(NBUF):
                    j = jj * NBUF + b
                    bn = (b + 1) % NBUF
                    s_wait(bn, j + 1 - NBUF)
                    g_issue(bn, (j + 1) % K)
                    g_wait(b, j)
                    s_issue(b, j)
                return carry

            lax.fori_loop(1, K // NBUF, ring, 0)
            for b in range(1, NBUF):
                s_wait(b, K - NBUF + b)
            g_wait(0, 0)
        plsc.subcore_barrier()
        pltpu.sync_copy(
            acc.at[pl.ds(s * ZROWS, ZROWS)],
            out.at[pl.ds(c * ACC_ROWS + s * ZROWS, ZROWS)])

    return spmm


def _make_sc_deg():
    @functools.partial(
        pl.kernel,
        out_type=jax.ShapeDtypeStruct((NC * ACC_ROWS, DEG_W), jnp.float32),
        mesh=_sc_mesh(),
        scratch_types=[
            pltpu.VMEM((K_CHUNKS, CHUNK), jnp.int32),
            pltpu.VMEM((CHUNK, DEG_W), jnp.float32),
            pltpu.VMEM_SHARED((ACC_ROWS, DEG_W), jnp.float32),
            pltpu.SemaphoreType.DMA,
        ],
    )
    def degk(col3, zeros, ones, out, col_v, ones_v, acc, sem):
        c = lax.axis_index("c")
        s = lax.axis_index("s")
        wid = s * NC + c
        pltpu.sync_copy(zeros.at[pl.ds(s * ZROWS, ZROWS)],
                        acc.at[pl.ds(s * ZROWS, ZROWS)])
        pltpu.sync_copy(col3.at[wid], col_v)
        pltpu.sync_copy(ones, ones_v)
        plsc.subcore_barrier()

        def group(jj, carry):
            base = jj * 8
            sds = [pltpu.async_copy(ones_v, acc.at[col_v.at[base + b]],
                                    sem, add=True) for b in range(8)]
            for sd in sds:
                sd.wait()
            return carry

        lax.fori_loop(0, K_CHUNKS // 8, group, 0)
        plsc.subcore_barrier()
        pltpu.sync_copy(
            acc.at[pl.ds(s * ZROWS, ZROWS)],
            out.at[pl.ds(c * ACC_ROWS + s * ZROWS, ZROWS)])

    return degk


_sc_spmm = _make_sc_spmm()
_sc_deg = _make_sc_deg()


def _tc_first(x, w0, deg):
    def body(x_ref, w_ref, deg_ref, o0, ou):
        dis = lax.rsqrt(deg_ref[...])
        o0[...] = jnp.dot(x_ref[...], w_ref[...],
                          preferred_element_type=jnp.float32)
        ou[...] = dis * x_ref[...]

    grid = (N_NODES // BLK,)
    return pl.pallas_call(
        body,
        grid=grid,
        in_specs=[
            pl.BlockSpec((BLK, D_IN), lambda i: (i, 0)),
            pl.BlockSpec((D_IN, HID), lambda i: (0, 0)),
            pl.BlockSpec((BLK, 1), lambda i: (i, 0)),
        ],
        out_specs=[
            pl.BlockSpec((BLK, HID), lambda i: (i, 0)),
            pl.BlockSpec((BLK, HID), lambda i: (i, 0)),
        ],
        out_shape=[jax.ShapeDtypeStruct((N_NODES, HID), jnp.float32)] * 2,
    )(x, w0, deg)


def _tc_mid1(sa, sb, u0, W12, deg):
    def body(a_ref, b_ref, u_ref, w_ref, deg_ref, o1, o2):
        dis = lax.rsqrt(deg_ref[...])
        x1 = dis * (a_ref[...] + b_ref[...] + u_ref[...])
        t = jnp.dot(x1, w_ref[...], preferred_element_type=jnp.float32)
        o1[...] = t[:, :HID]
        o2[...] = dis * t[:, HID:]

    grid = (N_NODES // BLK,)
    bs = pl.BlockSpec((BLK, HID), lambda i: (i, 0))
    return pl.pallas_call(
        body,
        grid=grid,
        in_specs=[bs, bs, bs,
                  pl.BlockSpec((HID, 2 * HID), lambda i: (0, 0)),
                  pl.BlockSpec((BLK, 1), lambda i: (i, 0))],
        out_specs=[bs, bs],
        out_shape=[jax.ShapeDtypeStruct((N_NODES, HID), jnp.float32)] * 2,
    )(sa, sb, u0, W12, deg)


def _tc_c1(o0, o1, s2a, s2b, u2p, bias, deg):
    def body(o0_ref, o1_ref, a2, b2, u2_ref, bias_ref, deg_ref,
             pre_ref, sums_ref):
        i = pl.program_id(0)
        dis = lax.rsqrt(deg_ref[...])
        out2 = dis * (a2[...] + b2[...] + u2_ref[...])
        pre = jnp.concatenate([o0_ref[...], o1_ref[...], out2],
                              axis=1) + bias_ref[...]
        pre_ref[...] = pre
        cs = jnp.concatenate(
            [jnp.sum(pre, axis=0, keepdims=True),
             jnp.sum(pre * pre, axis=0, keepdims=True)], axis=0)
        sums_ref[...] = jnp.where(i == 0, cs, sums_ref[...] + cs)

    grid = (N_NODES // BLK,)
    bs = pl.BlockSpec((BLK, HID), lambda i: (i, 0))
    return pl.pallas_call(
        body,
        grid=grid,
        in_specs=[bs, bs, bs, bs, bs,
                  pl.BlockSpec((1, H3), lambda i: (0, 0)),
                  pl.BlockSpec((BLK, 1), lambda i: (i, 0))],
        out_specs=[pl.BlockSpec((BLK, H3), lambda i: (i, 0)),
                   pl.BlockSpec((2, H3), lambda i: (0, 0))],
        out_shape=[jax.ShapeDtypeStruct((N_NODES, H3), jnp.float32),
                   jax.ShapeDtypeStruct((2, H3), jnp.float32)],
    )(o0, o1, s2a, s2b, u2p, bias, deg)


def _tc_a_bn(pre, sums, g, b, W, deg):
    def body(p_ref, s_ref, g_ref, b_ref, w_ref, deg_ref, o0, o1, o2):
        mu = s_ref[0:1, :] * (1.0 / N_NODES)
        var = s_ref[1:2, :] * (1.0 / N_NODES) - mu * mu
        inv = lax.rsqrt(var + EPS)
        h = (p_ref[...] - mu) * inv * g_ref[...] + b_ref[...]
        t = jnp.dot(h, w_ref[...], preferred_element_type=jnp.float32)
        dis = lax.rsqrt(deg_ref[...])
        o0[...] = t[:, :HID]
        o1[...] = dis * t[:, HID:2 * HID]
        o2[...] = dis * t[:, 2 * HID:]

    grid = (N_NODES // BLK,)
    return pl.pallas_call(
        body,
        grid=grid,
        in_specs=[
            pl.BlockSpec((BLK, H3), lambda i: (i, 0)),
            pl.BlockSpec((2, H3), lambda i: (0, 0)),
            pl.BlockSpec((1, H3), lambda i: (0, 0)),
            pl.BlockSpec((1, H3), lambda i: (0, 0)),
            pl.BlockSpec((H3, H3), lambda i: (0, 0)),
            pl.BlockSpec((BLK, 1), lambda i: (i, 0)),
        ],
        out_specs=[
            pl.BlockSpec((BLK, HID), lambda i: (i, 0)),
            pl.BlockSpec((BLK, HID), lambda i: (i, 0)),
            pl.BlockSpec((BLK, HID), lambda i: (i, 0)),
        ],
        out_shape=[jax.ShapeDtypeStruct((N_NODES, HID), jnp.float32)] * 3,
    )(pre, sums, g, b, W, deg)


def _tc_comb(sa, sb, u, deg):
    def body(a_ref, b_ref, u_ref, deg_ref, o_ref):
        o_ref[...] = (a_ref[...] + b_ref[...] + u_ref[...]) / deg_ref[...]

    grid = (N_NODES // BLK,)
    bs = pl.BlockSpec((BLK, HID), lambda i: (i, 0))
    return pl.pallas_call(
        body,
        grid=grid,
        in_specs=[bs, bs, bs, pl.BlockSpec((BLK, 1), lambda i: (i, 0))],
        out_specs=bs,
        out_shape=jax.ShapeDtypeStruct((N_NODES, HID), jnp.float32),
    )(sa, sb, u, deg)


def _tc_c(o0, s1a, s1b, u1, s2a, s2b, u2b, bias, deg):
    def body(o0_ref, a1, b1, u1_ref, a2, b2, u2_ref, bias_ref, deg_ref,
             pre_ref, sums_ref):
        i = pl.program_id(0)
        dis = lax.rsqrt(deg_ref[...])
        out1 = dis * (a1[...] + b1[...] + u1_ref[...])
        out2 = dis * (a2[...] + b2[...] + u2_ref[...])
        pre = jnp.concatenate([o0_ref[...], out1, out2], axis=1) + bias_ref[...]
        pre_ref[...] = pre
        cs = jnp.concatenate(
            [jnp.sum(pre, axis=0, keepdims=True),
             jnp.sum(pre * pre, axis=0, keepdims=True)], axis=0)
        sums_ref[...] = jnp.where(i == 0, cs, sums_ref[...] + cs)

    grid = (N_NODES // BLK,)
    bs = pl.BlockSpec((BLK, HID), lambda i: (i, 0))
    return pl.pallas_call(
        body,
        grid=grid,
        in_specs=[bs, bs, bs, bs, bs, bs, bs,
                  pl.BlockSpec((1, H3), lambda i: (0, 0)),
                  pl.BlockSpec((BLK, 1), lambda i: (i, 0))],
        out_specs=[pl.BlockSpec((BLK, H3), lambda i: (i, 0)),
                   pl.BlockSpec((2, H3), lambda i: (0, 0))],
        out_shape=[jax.ShapeDtypeStruct((N_NODES, H3), jnp.float32),
                   jax.ShapeDtypeStruct((2, H3), jnp.float32)],
    )(o0, s1a, s1b, u1, s2a, s2b, u2b, bias, deg)


def _tc_head(pre, sums, g, b, lw, lb):
    def body(p_ref, s_ref, g_ref, b_ref, w_ref, lb_ref, o_ref):
        mu = s_ref[0:1, :] * (1.0 / N_NODES)
        var = s_ref[1:2, :] * (1.0 / N_NODES) - mu * mu
        inv = lax.rsqrt(var + EPS)
        h = (p_ref[...] - mu) * inv * g_ref[...] + b_ref[...]
        logits = jnp.dot(h, w_ref[...], preferred_element_type=jnp.float32)
        logits = logits + lb_ref[...]
        m = jnp.max(logits, axis=1, keepdims=True)
        z = logits - m
        lse = jnp.log(jnp.sum(jnp.exp(z), axis=1, keepdims=True))
        o_ref[...] = z - lse

    grid = (N_NODES // BLK,)
    return pl.pallas_call(
        body,
        grid=grid,
        in_specs=[
            pl.BlockSpec((BLK, H3), lambda i: (i, 0)),
            pl.BlockSpec((2, H3), lambda i: (0, 0)),
            pl.BlockSpec((1, H3), lambda i: (0, 0)),
            pl.BlockSpec((1, H3), lambda i: (0, 0)),
            pl.BlockSpec((H3, OUT), lambda i: (0, 0)),
            pl.BlockSpec((1, OUT), lambda i: (0, 0)),
        ],
        out_specs=pl.BlockSpec((BLK, OUT), lambda i: (i, 0)),
        out_shape=jax.ShapeDtypeStruct((N_NODES, OUT), jnp.float32),
    )(pre, sums, g, b, lw, lb)


def kernel(x, edge_index, params):
    p = params
    row = edge_index[0]
    col = edge_index[1]
    npad = E_PAD - N_EDGES
    pad_r = (jnp.arange(npad, dtype=jnp.int32) % 128)
    pad_c = TRASH + (jnp.arange(npad, dtype=jnp.int32) % (ACC_ROWS - TRASH))
    row3 = (jnp.concatenate([row, pad_r])
            .reshape(K_CHUNKS, NW, CHUNK).transpose(1, 0, 2))
    col3 = (jnp.concatenate([col, pad_c])
            .reshape(K_CHUNKS, NW, CHUNK).transpose(1, 0, 2))
    zeros128 = jnp.zeros((ACC_ROWS, HID), jnp.float32)
    zeros16 = jnp.zeros((ACC_ROWS, DEG_W), jnp.float32)
    ones16 = jnp.ones((CHUNK, DEG_W), jnp.float32)

    degp = _sc_deg(col3, zeros16, ones16)
    deg = (degp[:N_NODES, 0]
           + degp[ACC_ROWS:ACC_ROWS + N_NODES, 0] + 1.0).reshape(N_NODES, 1)

    def S(u):
        sp = _sc_spmm(u, row3, col3, zeros128)
        return sp[:N_NODES], sp[ACC_ROWS:ACC_ROWS + N_NODES]

    out0, u0 = _tc_first(x, p["c1_w0"], deg)
    sa, sb = S(u0)
    W12 = jnp.concatenate([p["c1_w1"], p["c1_w2"]], axis=1)
    out1, u2p = _tc_mid1(sa, sb, u0, W12, deg)
    s2a, s2b = S(u2p)
    pre1, sums1 = _tc_c1(out0, out1, s2a, s2b, u2p,
                         p["c1_b"].reshape(1, H3), deg)

    pre, sums = pre1, sums1
    for l, nk in (("c2", "n1"), ("c3", "n2")):
        W = jnp.concatenate([p[l + "_w0"], p[l + "_w1"], p[l + "_w2"]], axis=1)
        out0, u1, u2 = _tc_a_bn(pre, sums, p[nk + "_g"].reshape(1, H3),
                                p[nk + "_b"].reshape(1, H3), W, deg)
        s1a, s1b = S(u1)
        s2a, s2b = S(u2)
        u2b = _tc_comb(s2a, s2b, u2, deg)
        s2ba, s2bb = S(u2b)
        pre, sums = _tc_c(out0, s1a, s1b, u1, s2ba, s2bb, u2b,
                          p[l + "_b"].reshape(1, H3), deg)

    return _tc_head(pre, sums, p["n3_g"].reshape(1, H3),
                    p["n3_b"].reshape(1, H3), p["lin_w"],
                    p["lin_b"].reshape(1, OUT))
